# baseline (speedup 1.0000x reference)
import functools

import jax
import jax.numpy as jnp
from jax import lax
from jax.experimental import pallas as pl
from jax.experimental.pallas import tpu as pltpu
from jax.experimental.pallas import tpu_sc as plsc


def kernel(r_idx, r_weight, kv):
    n, v, p2, w2, c_kv = kv.shape
    topk = r_idx.shape[-1]
    D = w2 * c_kv
    B = n * p2 * topk
    vp2 = v * p2

    table = kv.reshape(n * vp2, D)
    idx_flat = r_idx.reshape(B)

    ACTIVE = 16
    b_per_w = B // ACTIVE
    w_per_batch = ACTIVE // n

    mesh = plsc.VectorSubcoreMesh(core_axis_name="c", subcore_axis_name="s")

    @functools.partial(
        pl.kernel,
        mesh=mesh,
        out_type=jax.ShapeDtypeStruct((B, D), jnp.float32),
        scratch_types=[
            pltpu.VMEM((b_per_w,), jnp.int32),
            pltpu.VMEM((b_per_w, D), jnp.float32),
            pltpu.SemaphoreType.DMA,
        ],
    )
    def gather_kernel(table_hbm, idx_hbm, out_hbm, idx_v, rows_v, sem):
        wid = lax.axis_index("s") * 2 + lax.axis_index("c")

        @pl.when(wid < ACTIVE)
        def _():
            base = wid * b_per_w
            pltpu.sync_copy(idx_hbm.at[pl.ds(base, b_per_w)], idx_v)
            idx_v[...] = idx_v[...] + (wid // w_per_batch) * vp2
            pltpu.async_copy(table_hbm.at[idx_v], rows_v, sem).wait()
            pltpu.sync_copy(rows_v, out_hbm.at[pl.ds(base, b_per_w)])

    out = gather_kernel(table, idx_flat)
    return out.reshape(n, p2, topk, w2, c_kv)


# speedup vs baseline: 8.7419x; 8.7419x over previous
import functools

import jax
import jax.numpy as jnp
from jax import lax
from jax.experimental import pallas as pl
from jax.experimental.pallas import tpu as pltpu
from jax.experimental.pallas import tpu_sc as plsc


def kernel(r_idx, r_weight, kv):
    n, v, p2, w2, c_kv = kv.shape
    topk = r_idx.shape[-1]
    B = n * p2 * topk
    vp2 = v * p2

    table = kv.reshape(n * vp2, w2, c_kv)
    idx_flat = r_idx.reshape(B)

    ACTIVE = 16
    b_per_w = B // ACTIVE
    w_per_batch = ACTIVE // n

    mesh = plsc.VectorSubcoreMesh(core_axis_name="c", subcore_axis_name="s")

    @functools.partial(
        pl.kernel,
        mesh=mesh,
        out_type=jax.ShapeDtypeStruct((B, w2, c_kv), jnp.float32),
        scratch_types=[
            pltpu.VMEM((b_per_w,), jnp.int32),
            pltpu.VMEM((b_per_w, w2, c_kv), jnp.float32),
            pltpu.SemaphoreType.DMA,
        ],
    )
    def gather_kernel(table_hbm, idx_hbm, out_hbm, idx_v, rows_v, sem):
        wid = lax.axis_index("s") * 2 + lax.axis_index("c")

        @pl.when(wid < ACTIVE)
        def _():
            base = wid * b_per_w
            pltpu.sync_copy(idx_hbm.at[pl.ds(base, b_per_w)], idx_v)
            idx_v[...] = idx_v[...] + (wid // w_per_batch) * vp2
            pltpu.async_copy(table_hbm.at[idx_v], rows_v, sem).wait()
            pltpu.sync_copy(rows_v, out_hbm.at[pl.ds(base, b_per_w)])

    out = gather_kernel(table, idx_flat)
    return out.reshape(n, p2, topk, w2, c_kv)


# speedup vs baseline: 9.5407x; 1.0914x over previous
import functools

import jax
import jax.numpy as jnp
from jax import lax
from jax.experimental import pallas as pl
from jax.experimental.pallas import tpu as pltpu
from jax.experimental.pallas import tpu_sc as plsc


def kernel(r_idx, r_weight, kv):
    n, v, p2, w2, c_kv = kv.shape
    topk = r_idx.shape[-1]
    B = n * p2 * topk
    vp2 = v * p2

    table = kv.reshape(n, vp2, w2, c_kv)

    NW = 32
    b_per_w = B // NW
    w_per_batch = NW // n
    w_per_row = topk // b_per_w

    mesh = plsc.VectorSubcoreMesh(core_axis_name="c", subcore_axis_name="s")

    @functools.partial(
        pl.kernel,
        mesh=mesh,
        out_type=jax.ShapeDtypeStruct((B, w2, c_kv), jnp.float32),
        scratch_types=[
            pltpu.VMEM((b_per_w,), jnp.int32),
            pltpu.VMEM((b_per_w, w2, c_kv), jnp.float32),
            pltpu.SemaphoreType.DMA,
        ],
    )
    def gather_kernel(table_hbm, idx_hbm, out_hbm, idx_v, rows_v, sem):
        wid = lax.axis_index("s") * 2 + lax.axis_index("c")
        b = wid // w_per_batch
        i = (wid % w_per_batch) // w_per_row
        k0 = (wid % w_per_row) * b_per_w
        pltpu.sync_copy(idx_hbm.at[b, i, pl.ds(k0, b_per_w)], idx_v)
        pltpu.async_copy(table_hbm.at[b].at[idx_v], rows_v, sem).wait()
        pltpu.sync_copy(rows_v, out_hbm.at[pl.ds(wid * b_per_w, b_per_w)])

    out = gather_kernel(table, r_idx)
    return out.reshape(n, p2, topk, w2, c_kv)
